# Initial kernel scaffold; baseline (speedup 1.0000x reference)
import jax
import jax.numpy as jnp
from jax.experimental import pallas as pl


def kernel(x, edge_index, batch, cls_embed, mean_embed, params):
    raise NotImplementedError("write your pallas kernel here")


# speedup vs baseline: 17.0459x; 17.0459x over previous
import functools

import jax
import jax.numpy as jnp
from jax import lax
from jax.experimental import pallas as pl
from jax.experimental.pallas import tpu as pltpu
from jax.experimental.pallas import tpu_sc as plsc

N = 10000
E = 320000
NPAD = 10240
NGRAPH = 256
NC, NS, LANES = 2, 16, 16
NW = NC * NS
EPT = E // NW
CHUNK = 80
NCHUNKS = EPT // CHUNK

_HI = jax.lax.Precision.HIGHEST
_CP = pltpu.CompilerParams(needs_layout_passes=False, use_tc_tiling_on_sc=False)
_MESH = plsc.VectorSubcoreMesh(
    core_axis_name="c", subcore_axis_name="s", num_cores=NC, num_subcores=NS)


@functools.cache
def _sc_plain(C: int, H: int):
    RPT = H // NS
    scratch = [
        pltpu.VMEM_SHARED((H, C), jnp.float32),
        pltpu.VMEM((CHUNK,), jnp.int32),
        pltpu.VMEM((CHUNK,), jnp.int32),
        pltpu.VMEM((CHUNK, C), jnp.float32),
        pltpu.SemaphoreType.DMA,
    ]
    out_type = jax.ShapeDtypeStruct((NC, H, C), jnp.float32)

    def body(table, src, dst, out, acc, src_v, dst_v, rows_v, sem):
        cid = lax.axis_index("c")
        sid = lax.axis_index("s")
        wid = sid * NC + cid
        for k in range(RPT // 160):
            pltpu.sync_copy(table.at[pl.ds(N, 160)],
                            acc.at[pl.ds(sid * RPT + k * 160, 160)])
        plsc.subcore_barrier()

        def chunk_body(i, carry):
            base = wid * EPT + i * CHUNK
            pltpu.sync_copy(src.at[pl.ds(base, CHUNK)], src_v)
            pltpu.sync_copy(dst.at[pl.ds(base, CHUNK)], dst_v)
            pltpu.async_copy(table.at[src_v], rows_v, sem).wait()
            pltpu.sync_copy(rows_v, acc.at[dst_v], add=True)
            return carry

        lax.fori_loop(0, NCHUNKS, chunk_body, 0)
        plsc.subcore_barrier()
        pltpu.sync_copy(acc.at[pl.ds(sid * RPT, RPT)],
                        out.at[cid, pl.ds(sid * RPT, RPT)])

    return pl.kernel(body, out_type=out_type, mesh=_MESH,
                     scratch_types=scratch, compiler_params=_CP)


@functools.cache
def _sc_index():
    scratch = [
        pltpu.VMEM((CHUNK,), jnp.int32),
        pltpu.VMEM((CHUNK,), jnp.int32),
        pltpu.VMEM((CHUNK,), jnp.int32),
        pltpu.VMEM((CHUNK,), jnp.int32),
        pltpu.VMEM((NPAD,), jnp.float32),
        pltpu.VMEM((NPAD,), jnp.float32),
    ]
    out_type = [jax.ShapeDtypeStruct((E,), jnp.int32),
                jax.ShapeDtypeStruct((E,), jnp.int32)]

    def body(src, dst, als, ald, osrc, odst, src_v, dst_v, src2_v, dst2_v,
             als_v, ald_v):
        cid = lax.axis_index("c")
        sid = lax.axis_index("s")
        wid = sid * NC + cid
        pltpu.sync_copy(als, als_v)
        pltpu.sync_copy(ald, ald_v)

        def chunk_body(i, carry):
            base = wid * EPT + i * CHUNK
            pltpu.sync_copy(src.at[pl.ds(base, CHUNK)], src_v)
            pltpu.sync_copy(dst.at[pl.ds(base, CHUNK)], dst_v)
            for j in range(CHUNK // LANES):
                sl = pl.ds(j * LANES, LANES)
                s_ = src_v[sl]
                d_ = dst_v[sl]
                a = plsc.load_gather(als_v, [s_])
                b = plsc.load_gather(ald_v, [d_])
                off = jnp.where(a + b > 0.0, 0, NPAD).astype(jnp.int32)
                src2_v[sl] = s_ + off
                dst2_v[sl] = d_ + off
            pltpu.sync_copy(src2_v, osrc.at[pl.ds(base, CHUNK)])
            pltpu.sync_copy(dst2_v, odst.at[pl.ds(base, CHUNK)])
            return carry

        lax.fori_loop(0, NCHUNKS, chunk_body, 0)

    return pl.kernel(body, out_type=out_type, mesh=_MESH,
                     scratch_types=scratch, compiler_params=_CP)


BN = 2048
GRID = NPAD // BN


def _bs(a):
    if a.ndim == 3 and a.shape[1] == NPAD:
        return pl.BlockSpec((2, BN, a.shape[2]), lambda i: (0, i, 0))
    if a.ndim == 2 and a.shape[0] in (N, NPAD):
        return pl.BlockSpec((BN, a.shape[1]), lambda i: (i, 0))
    return pl.BlockSpec(a.shape, lambda i: (0,) * a.ndim)


def _mm(xs, W, b=None, act=False, pre=None):
    M = W.shape[1]
    args = list(xs) + [W] + ([b] if b is not None else [])

    def body(*refs):
        n_x = len(xs)
        xrefs = refs[:n_x]
        Wr = refs[n_x]
        br = refs[n_x + 1] if b is not None else None
        o = refs[-1]
        xb = pre(*(r[...] for r in xrefs)) if pre else xrefs[0][...]
        y = jnp.dot(xb, Wr[...], precision=_HI, preferred_element_type=jnp.float32)
        if br is not None:
            y = y + br[...]
        if act:
            y = jnp.maximum(y, 0.0)
        o[...] = y

    return pl.pallas_call(
        body,
        grid=(GRID,),
        in_specs=[_bs(a) for a in args],
        out_specs=pl.BlockSpec((BN, M), lambda i: (i, 0)),
        out_shape=jax.ShapeDtypeStruct((N, M), jnp.float32),
    )(*args)


def _gat_pre(xs, W, A, C, pre=None):
    Ch = W.shape[1]

    def body(*refs):
        n_x = len(xs)
        xrefs = refs[:n_x]
        Wr, Ar = refs[n_x], refs[n_x + 1]
        ho, alo, tpo, tno = refs[-4:]
        xb = pre(*(r[...] for r in xrefs)) if pre else xrefs[0][...]
        h = jnp.dot(xb, Wr[...], precision=_HI, preferred_element_type=jnp.float32)
        ho[...] = h
        al = jnp.dot(h, Ar[...], precision=_HI,
                     preferred_element_type=jnp.float32)
        alo[...] = al
        z = jnp.zeros((BN, C - Ch - 1), jnp.float32)
        one = jnp.ones((BN, 1), jnp.float32)
        ws = jnp.exp(al[:, 0:1])
        tpo[...] = jnp.concatenate([ws * h, ws * one, z], axis=1)
        ws2 = jnp.exp(0.2 * al[:, 0:1])
        tno[...] = jnp.concatenate([ws2 * h, ws2 * one, z], axis=1)

    args = list(xs) + [W, A]
    return pl.pallas_call(
        body,
        grid=(GRID,),
        in_specs=[_bs(a) for a in args],
        out_specs=[pl.BlockSpec((BN, Ch), lambda i: (i, 0)),
                   pl.BlockSpec((BN, 2), lambda i: (i, 0)),
                   pl.BlockSpec((BN, C), lambda i: (i, 0)),
                   pl.BlockSpec((BN, C), lambda i: (i, 0))],
        out_shape=[jax.ShapeDtypeStruct((N, Ch), jnp.float32),
                   jax.ShapeDtypeStruct((N, 2), jnp.float32),
                   jax.ShapeDtypeStruct((N, C), jnp.float32),
                   jax.ShapeDtypeStruct((N, C), jnp.float32)],
    )(*args)


def _gat_post(partials, h, al, bias, relu=False, res=None):
    Ch = h.shape[1]
    C = partials.shape[2]
    OFF = NPAD // BN
    args = [partials, partials, h, al, bias] + ([res] if res is not None else [])

    def body(*refs):
        pP, pN, hr, alr, br = refs[0], refs[1], refs[2], refs[3], refs[4]
        rr = refs[5] if res is not None else None
        o = refs[-1]
        accP = pP[0] + pP[1]
        accN = pN[0] + pN[1]
        ed = jnp.exp(alr[:, 1:2])
        ed2 = jnp.exp(0.2 * alr[:, 1:2])
        num = ed * accP[:, :Ch] + ed2 * accN[:, :Ch]
        den = ed * accP[:, Ch:Ch + 1] + ed2 * accN[:, Ch:Ch + 1]
        el = alr[:, 0:1] + alr[:, 1:2]
        wl = jnp.exp(jnp.maximum(el, el * 0.2))
        y = (num + wl * hr[...]) / (den + wl) + br[...]
        if res is not None:
            y = y + rr[...]
        if relu:
            y = jnp.maximum(y, 0.0)
        o[...] = y

    in_specs = [pl.BlockSpec((2, BN, C), lambda i: (0, i, 0)),
                pl.BlockSpec((2, BN, C), lambda i: (0, i + OFF, 0))]
    in_specs += [_bs(a) for a in args[2:]]
    return pl.pallas_call(
        body,
        grid=(GRID,),
        in_specs=in_specs,
        out_specs=pl.BlockSpec((BN, Ch), lambda i: (i, 0)),
        out_shape=jax.ShapeDtypeStruct((N, Ch), jnp.float32),
    )(*args)


def _mf(pagg, xin, Wl, bl, Wr, deg=None, relu=False):
    Kin = Wl.shape[1]
    M = Wl.shape[2]
    layer1 = deg is None
    args = [pagg, xin, Wl, bl, Wr] + ([] if layer1 else [deg])

    def body(*refs):
        pr, xr, Wlr, blr, Wrr = refs[:5]
        o = refs[-2] if layer1 else refs[-1]
        aug = pr[0] + pr[1]
        agg = aug[:, :Kin]
        if layer1:
            degf = aug[:, 128:129]
        else:
            degf = refs[5][...]
        di = jnp.clip(degf.astype(jnp.int32), 0, 10)
        acc = jnp.zeros((BN, M), jnp.float32)
        for d in range(11):
            mask = (di == d).astype(jnp.float32)
            term = (jnp.dot(agg, Wlr[d], precision=_HI,
                            preferred_element_type=jnp.float32)
                    + blr[d]
                    + jnp.dot(xr[...], Wrr[d], precision=_HI,
                              preferred_element_type=jnp.float32))
            acc = acc + mask * term
        if relu:
            acc = jnp.maximum(acc, 0.0)
        o[...] = acc
        if layer1:
            refs[-1][...] = degf

    out_specs = [pl.BlockSpec((BN, M), lambda i: (i, 0))]
    out_shape = [jax.ShapeDtypeStruct((N, M), jnp.float32)]
    if layer1:
        out_specs.append(pl.BlockSpec((BN, 1), lambda i: (i, 0)))
        out_shape.append(jax.ShapeDtypeStruct((N, 1), jnp.float32))
    outs = pl.pallas_call(
        body,
        grid=(GRID,),
        in_specs=[_bs(a) for a in args],
        out_specs=out_specs if layer1 else out_specs[0],
        out_shape=out_shape if layer1 else out_shape[0],
    )(*args)
    return outs


def _pool(Haug, batchcol):
    CW = Haug.shape[1]

    def body(hr, br, o):
        i = pl.program_id(0)

        @pl.when(i == 0)
        def _():
            o[...] = jnp.zeros((NGRAPH, CW), jnp.float32)

        gids = lax.broadcasted_iota(jnp.int32, (BN, NGRAPH), 1)
        oh = (br[...] == gids).astype(jnp.float32)
        o[...] += lax.dot_general(
            oh, hr[...], (((0,), (0,)), ((), ())),
            precision=_HI, preferred_element_type=jnp.float32)

    return pl.pallas_call(
        body,
        grid=(GRID,),
        in_specs=[pl.BlockSpec((BN, CW), lambda i: (i, 0)),
                  pl.BlockSpec((BN, 1), lambda i: (i, 0))],
        out_specs=pl.BlockSpec((NGRAPH, CW), lambda i: (0, 0)),
        out_shape=jax.ShapeDtypeStruct((NGRAPH, CW), jnp.float32),
    )(Haug, batchcol)


def _head(pooled, cls_embed, mean_embed, m, att, fl):
    weights = [
        m['cW1'], m['cb1'], m['cW2'], m['cb2'], m['cW3'], m['cb3'],
        m['mW1'], m['mb1'], m['mW2'], m['mb2'], m['mW3'], m['mb3'],
        m['fW1'], m['fb1'], m['fW2'], m['fb2'], m['fW3'], m['fb3'],
        att['gW1'], att['gb1'], att['gW2'], att['gb2'],
        fl['W1'], fl['b1'], fl['W2'], fl['b2'],
    ]
    weights = [w.reshape(1, -1) if w.ndim == 1 else w for w in weights]
    args = [pooled, cls_embed, mean_embed] + weights

    def body(*refs):
        (po, cl, me,
         cW1, cb1, cW2, cb2, cW3, cb3,
         mW1, mb1, mW2, mb2, mW3, mb3,
         fW1, fb1, fW2, fb2, fW3, fb3,
         gW1, gb1, gW2, gb2,
         W1, b1, W2, b2, o) = refs

        def lin(a, W, b_, act=False):
            y = jnp.dot(a, W[...], precision=_HI,
                        preferred_element_type=jnp.float32) + b_[...]
            return jnp.maximum(y, 0.0) if act else y

        pooledv = po[...]
        cnt = jnp.maximum(pooledv[:, 256:257], 1.0)
        branches = [pooledv[:, 0:64] / cnt, pooledv[:, 64:128] / cnt,
                    pooledv[:, 128:192] / cnt, pooledv[:, 192:256] / cnt]
        c = lin(cl[...], cW1, cb1, True)
        c = lin(c, cW2, cb2, True)
        c = lin(c, cW3, cb3)
        mm_ = lin(me[...], mW1, mb1, True)
        mm_ = lin(mm_, mW2, mb2, True)
        mm_ = lin(mm_, mW3, mb3)
        comb = jnp.concatenate([c, mm_], axis=1)
        f = lin(comb, fW1, fb1, True)
        f = lin(f, fW2, fb2, True)
        branches.append(lin(f, fW3, fb3))

        gates = [lin(jnp.maximum(lin(br, gW1, gb1), 0.0), gW2, gb2)
                 for br in branches]
        gmax = gates[0]
        for g_ in gates[1:]:
            gmax = jnp.maximum(gmax, g_)
        exps = [jnp.exp(g_ - gmax) for g_ in gates]
        ssum = exps[0] + exps[1] + exps[2] + exps[3] + exps[4]
        agg = jnp.zeros((NGRAPH, 64), jnp.float32)
        for e_, br in zip(exps, branches):
            agg = agg + (e_ / ssum) * br
        y = jnp.maximum(lin(agg, W1, b1), 0.0)
        o[...] = lin(y, W2, b2)

    return pl.pallas_call(
        body,
        grid=(1,),
        in_specs=[pl.BlockSpec(a.shape, lambda i: (0,) * a.ndim) for a in args],
        out_specs=pl.BlockSpec((NGRAPH, 1), lambda i: (0, 0)),
        out_shape=jax.ShapeDtypeStruct((NGRAPH, 1), jnp.float32),
    )(*args)


def _pad_rows(a, H=NPAD):
    return jnp.zeros((H, a.shape[1]), a.dtype).at[:a.shape[0]].set(a)


def _pad_vec(v):
    return jnp.zeros((NPAD,), jnp.float32).at[:N].set(v)


def _gat_conv_sc(xs, W, a_s, a_d, bias, src, dst, relu=False, res=None,
                 pre=None):
    Ch = W.shape[1]
    C = Ch + 8
    A = jnp.stack([a_s, a_d], axis=1)
    h, al, tp, tn = _gat_pre(xs, W, A, C, pre=pre)
    T = jnp.concatenate([_pad_rows(tp), _pad_rows(tn)], axis=0)
    s2, d2 = _sc_index()(src, dst, _pad_vec(al[:, 0]), _pad_vec(al[:, 1]))
    partials = _sc_plain(C, 2 * NPAD)(T, s2, d2)
    return _gat_post(partials, h, al, bias.reshape(1, -1), relu=relu, res=res), h


def kernel(x, edge_index, batch, cls_embed, mean_embed, params):
    p = params
    src = edge_index[0]
    dst = edge_index[1]

    x_aug = _pad_rows(jnp.concatenate(
        [x, jnp.ones((N, 1), jnp.float32), jnp.zeros((N, 7), jnp.float32)],
        axis=1))
    p1 = _sc_plain(136, NPAD)(x_aug, src, dst)

    mf = p['mf']
    mf_h, degf = _mf(p1, x, mf['Wl1'], mf['bl1'].reshape(11, 1, -1),
                     mf['Wr1'], relu=True)
    p2 = _sc_plain(32, NPAD)(_pad_rows(mf_h), src, dst)
    mf_out = _mf(p2, mf_h, mf['Wl2'], mf['bl2'].reshape(11, 1, -1),
                 mf['Wr2'], deg=degf)

    gi = p['gin']
    t1 = _mm([x, p1], gi['W1a'], gi['b1a'].reshape(1, -1), act=True,
             pre=lambda xb, pb: xb + pb[0][:, :128] + pb[1][:, :128])
    h_gin1 = _mm([t1], gi['W1b'], gi['b1b'].reshape(1, -1), act=True)
    p3 = _sc_plain(128, NPAD)(_pad_rows(h_gin1), src, dst)
    t2 = _mm([h_gin1, p3], gi['W2a'], gi['b2a'].reshape(1, -1), act=True,
             pre=lambda hb, pb: hb + pb[0] + pb[1])
    h_gin2 = _mm([t2], gi['W2b'], gi['b2b'].reshape(1, -1), act=True)
    gin_out = _mm([h_gin2], gi['Wo'], gi['bo'].reshape(1, -1))

    g = p['gat']
    h_gat, _ = _gat_conv_sc([x], g['W1'], g['as1'], g['ad1'], g['b1'],
                            src, dst, relu=True)
    gat_out, _ = _gat_conv_sc([h_gat], g['W2'], g['as2'], g['ad2'], g['b2'],
                              src, dst)

    dg = p['dgat']
    h0, _ = _gat_conv_sc([x], dg['Wi'], dg['asi'], dg['adi'], dg['bi'],
                         src, dst)
    bn_scale = (dg['bn_g'] / jnp.sqrt(dg['bn_rv'] + 1e-5)).reshape(1, -1)
    bn_shift = (dg['bn_b'] - dg['bn_rm'].reshape(1, -1) * bn_scale)
    conv_m, _ = _gat_conv_sc(
        [h0, bn_scale, bn_shift], dg['Wm'], dg['asm'], dg['adm'], dg['bm'],
        src, dst, pre=lambda hb, sc_, sh_: jnp.maximum(hb * sc_ + sh_, 0.0))
    dgat_out, _ = _gat_conv_sc(
        [h0, conv_m], dg['Wo'], dg['aso'], dg['ado'], dg['bo'],
        src, dst, pre=lambda a, b_: a + b_)

    Haug = jnp.concatenate(
        [mf_out, gat_out, dgat_out, gin_out, jnp.ones((N, 64), jnp.float32)],
        axis=1)
    batchcol = jnp.full((NPAD, 1), NGRAPH + 7, jnp.int32).at[:N, 0].set(batch)
    pooled = _pool(_pad_rows(Haug), batchcol)
    return _head(pooled, cls_embed, mean_embed, p['mlp'], p['att'], p['final'])
